# baseline (speedup 1.0000x reference)
import functools

import jax
import jax.numpy as jnp
from jax import lax
from jax.experimental import pallas as pl
from jax.experimental.pallas import tpu as pltpu
from jax.experimental.pallas import tpu_sc as plsc

NUM_EMB = 1000000
DIM = 32
BATCH = 16384
SEQ = 50
TOTAL = BATCH * SEQ

NC = 2
NS = 16
NW = NC * NS
PER_W = TOTAL // NW
G = 128
STEPS = PER_W // G
GROUPS_PER_S = BATCH // G
OUT_ROWS = TOTAL * DIM // G

_mesh = plsc.VectorSubcoreMesh(core_axis_name="c", subcore_axis_name="s")


def _transpose_block(rows_v, tr_v):
    lanes = lax.iota(jnp.int32, 16)
    c_lo = lanes * G
    c_hi = (lanes + 16) * G

    @pl.loop(0, G, step=8)
    def _t(k0):
        for u in range(8):
            k = k0 + u
            kv = jnp.broadcast_to(k, (16,)).astype(jnp.int32)
            v_lo = rows_v[k, pl.ds(0, 16)]
            v_hi = rows_v[k, pl.ds(16, 16)]
            plsc.store_scatter(tr_v, [c_lo + kv], v_lo)
            plsc.store_scatter(tr_v, [c_hi + kv], v_hi)


@functools.partial(
    pl.kernel,
    out_type=jax.ShapeDtypeStruct((TOTAL * DIM,), jnp.float32),
    mesh=_mesh,
    scratch_types=[
        pltpu.VMEM((SEQ, BATCH // NW), jnp.int32),
        pltpu.VMEM((2, G, DIM), jnp.float32),
        pltpu.VMEM((2, DIM * G), jnp.float32),
        pltpu.SemaphoreType.DMA,
        pltpu.SemaphoreType.DMA,
    ],
    compiler_params=pltpu.CompilerParams(
        use_tc_tiling_on_sc=False, needs_layout_passes=False
    ),
)
def _emb_lookup(idx_hbm, table_hbm, out_hbm, idx_v, rows_v, tr_v, gsem, ssem):
    wid = lax.axis_index("s") * NC + lax.axis_index("c")
    bw = BATCH // NW
    jpw = bw // G
    pltpu.sync_copy(idx_hbm.at[:, pl.ds(wid * bw, bw)], idx_v)

    def start_gather(i, slot):
        pltpu.async_copy(
            table_hbm.at[idx_v.at[i // jpw, pl.ds((i % jpw) * G, G)]],
            rows_v.at[slot],
            gsem,
        )

    def wait_gather(slot):
        pltpu.make_async_copy(
            table_hbm.at[idx_v.at[0, pl.ds(0, G)]], rows_v.at[slot], gsem
        ).wait()

    def start_stores(i, slot):
        s = i // jpw
        tb = wid * jpw + i % jpw
        base = s * 524288 + tb * 1024
        for tf in range(4):
            pltpu.async_copy(
                tr_v.at[slot].at[pl.ds(tf * 1024, 1024)],
                out_hbm.at[pl.ds(base + tf * 131072, 1024)],
                ssem,
            )

    def wait_stores(slot):
        for tf in range(4):
            pltpu.make_async_copy(
                tr_v.at[slot].at[pl.ds(tf * 1024, 1024)],
                out_hbm.at[pl.ds(0, 1024)],
                ssem,
            ).wait()

    start_gather(0, 0)

    @pl.loop(0, STEPS, step=2)
    def _main(i):
        for slot in range(2):
            g = i + slot
            wait_gather(slot)

            @pl.when(g + 1 < STEPS)
            def _():
                start_gather(g + 1, 1 - slot)

            @pl.when(g >= 2)
            def _():
                wait_stores(slot)

            _transpose_block(rows_v.at[slot], tr_v.at[slot])
            start_stores(g, slot)

    wait_stores(0)
    wait_stores(1)


def kernel(token_ids, embedding_table):
    idx = jnp.transpose(token_ids).astype(jnp.int32)
    out_lin = _emb_lookup(idx, embedding_table)
    x = out_lin.reshape(SEQ, 4, GROUPS_PER_S, 8, G)
    x = jnp.transpose(x, (2, 4, 0, 1, 3))
    return x.reshape(BATCH, SEQ, DIM)


# speedup vs baseline: 1.0002x; 1.0002x over previous
import functools

import jax
import jax.numpy as jnp
from jax import lax
from jax.experimental import pallas as pl
from jax.experimental.pallas import tpu as pltpu
from jax.experimental.pallas import tpu_sc as plsc

NUM_EMB = 1000000
DIM = 32
BATCH = 16384
SEQ = 50
TOTAL = BATCH * SEQ

NC = 2
NS = 16
NW = NC * NS
PER_W = TOTAL // NW
G = 128
STEPS = PER_W // G
GROUPS_PER_S = BATCH // G
OUT_ROWS = TOTAL * DIM // G

_mesh = plsc.VectorSubcoreMesh(core_axis_name="c", subcore_axis_name="s")


def _transpose_block(rows_v, tr_v):
    lanes = lax.iota(jnp.int32, 16)
    cs = [lanes * G + u for u in range(8)]
    span = 15 * G + 8

    @pl.loop(0, G, step=8)
    def _t(k0):
        ref_lo = tr_v.at[pl.ds(k0, span)]
        ref_hi = tr_v.at[pl.ds(k0 + 16 * G, span)]
        for u in range(8):
            k = k0 + u
            v_lo = rows_v[k, pl.ds(0, 16)]
            v_hi = rows_v[k, pl.ds(16, 16)]
            plsc.store_scatter(ref_lo, [cs[u]], v_lo)
            plsc.store_scatter(ref_hi, [cs[u]], v_hi)


@functools.partial(
    pl.kernel,
    out_type=jax.ShapeDtypeStruct((TOTAL * DIM,), jnp.float32),
    mesh=_mesh,
    scratch_types=[
        pltpu.VMEM((SEQ, BATCH // NW), jnp.int32),
        pltpu.VMEM((2, G, DIM), jnp.float32),
        pltpu.VMEM((2, DIM * G), jnp.float32),
        pltpu.SemaphoreType.DMA,
        pltpu.SemaphoreType.DMA,
    ],
    compiler_params=pltpu.CompilerParams(
        use_tc_tiling_on_sc=False, needs_layout_passes=False
    ),
)
def _emb_lookup(idx_hbm, table_hbm, out_hbm, idx_v, rows_v, tr_v, gsem, ssem):
    wid = lax.axis_index("s") * NC + lax.axis_index("c")
    bw = BATCH // NW
    jpw = bw // G
    pltpu.sync_copy(idx_hbm.at[:, pl.ds(wid * bw, bw)], idx_v)

    def start_gather(i, slot):
        pltpu.async_copy(
            table_hbm.at[idx_v.at[i // jpw, pl.ds((i % jpw) * G, G)]],
            rows_v.at[slot],
            gsem,
        )

    def wait_gather(slot):
        pltpu.make_async_copy(
            table_hbm.at[idx_v.at[0, pl.ds(0, G)]], rows_v.at[slot], gsem
        ).wait()

    def start_stores(i, slot):
        s = i // jpw
        tb = wid * jpw + i % jpw
        base = s * 524288 + tb * 1024
        for tf in range(4):
            pltpu.async_copy(
                tr_v.at[slot].at[pl.ds(tf * 1024, 1024)],
                out_hbm.at[pl.ds(base + tf * 131072, 1024)],
                ssem,
            )

    def wait_stores(slot):
        for tf in range(4):
            pltpu.make_async_copy(
                tr_v.at[slot].at[pl.ds(tf * 1024, 1024)],
                out_hbm.at[pl.ds(0, 1024)],
                ssem,
            ).wait()

    start_gather(0, 0)

    @pl.loop(0, STEPS, step=2)
    def _main(i):
        for slot in range(2):
            g = i + slot
            wait_gather(slot)

            @pl.when(g + 1 < STEPS)
            def _():
                start_gather(g + 1, 1 - slot)

            @pl.when(g >= 2)
            def _():
                wait_stores(slot)

            _transpose_block(rows_v.at[slot], tr_v.at[slot])
            start_stores(g, slot)

    wait_stores(0)
    wait_stores(1)


def kernel(token_ids, embedding_table):
    idx = jnp.transpose(token_ids).astype(jnp.int32)
    out_lin = _emb_lookup(idx, embedding_table)
    x = out_lin.reshape(SEQ, 4, GROUPS_PER_S, 8, G)
    x = jnp.transpose(x, (2, 4, 0, 1, 3))
    return x.reshape(BATCH, SEQ, DIM)


# speedup vs baseline: 1.4533x; 1.4530x over previous
import functools

import jax
import jax.numpy as jnp
from jax import lax
from jax.experimental import pallas as pl
from jax.experimental.pallas import tpu as pltpu
from jax.experimental.pallas import tpu_sc as plsc

NUM_EMB = 1000000
DIM = 32
BATCH = 16384
SEQ = 50
TOTAL = BATCH * SEQ

NC = 2
NS = 16
NW = NC * NS
PER_W = TOTAL // NW
G = 128
STEPS = PER_W // G
GROUPS_PER_S = BATCH // G
OUT_ROWS = TOTAL * DIM // G
TR_STRIDE = 131

_mesh = plsc.VectorSubcoreMesh(core_axis_name="c", subcore_axis_name="s")


def _transpose_block(rows_v, tr_v):
    lanes = lax.iota(jnp.int32, 16)
    hi_lanes = lanes + 16

    @pl.loop(0, G, step=8)
    def _t(k0):
        for u in range(8):
            k = k0 + u
            kv = jnp.broadcast_to(k, (16,))
            v_lo = rows_v[k, pl.ds(0, 16)]
            v_hi = rows_v[k, pl.ds(16, 16)]
            plsc.store_scatter(tr_v, [lanes, kv], v_lo)
            plsc.store_scatter(tr_v, [hi_lanes, kv], v_hi)


@functools.partial(
    pl.kernel,
    out_type=jax.ShapeDtypeStruct((OUT_ROWS, G), jnp.float32),
    mesh=_mesh,
    scratch_types=[
        pltpu.VMEM((SEQ, BATCH // NW), jnp.int32),
        pltpu.VMEM((2, G, DIM), jnp.float32),
        pltpu.VMEM((2, DIM, TR_STRIDE), jnp.float32),
        pltpu.SemaphoreType.DMA,
        pltpu.SemaphoreType.DMA,
    ],
    compiler_params=pltpu.CompilerParams(
        use_tc_tiling_on_sc=False, needs_layout_passes=False
    ),
)
def _emb_lookup(idx_hbm, table_hbm, out_hbm, idx_v, rows_v, tr_v, gsem, ssem):
    wid = lax.axis_index("s") * NC + lax.axis_index("c")
    bw = BATCH // NW
    jpw = bw // G
    pltpu.sync_copy(idx_hbm.at[:, pl.ds(wid * bw, bw)], idx_v)

    def start_gather(i, slot):
        pltpu.async_copy(
            table_hbm.at[idx_v.at[i // jpw, pl.ds((i % jpw) * G, G)]],
            rows_v.at[slot],
            gsem,
        )

    def wait_gather(slot):
        pltpu.make_async_copy(
            table_hbm.at[idx_v.at[0, pl.ds(0, G)]], rows_v.at[slot], gsem
        ).wait()

    def start_stores(i, slot):
        s = i // jpw
        tb = wid * jpw + i % jpw
        base = s * 4096 + tb * 8
        for tf in range(4):
            pltpu.async_copy(
                tr_v.at[slot].at[pl.ds(8 * tf, 8), pl.ds(0, G)],
                out_hbm.at[pl.ds(base + tf * 1024, 8)],
                ssem,
            )

    def wait_stores(slot):
        for tf in range(4):
            pltpu.make_async_copy(
                tr_v.at[slot].at[pl.ds(8 * tf, 8), pl.ds(0, G)],
                out_hbm.at[pl.ds(0, 8)],
                ssem,
            ).wait()

    start_gather(0, 0)

    @pl.loop(0, STEPS, step=2)
    def _main(i):
        for slot in range(2):
            g = i + slot
            wait_gather(slot)

            @pl.when(g + 1 < STEPS)
            def _():
                start_gather(g + 1, 1 - slot)

            @pl.when(g >= 2)
            def _():
                wait_stores(slot)

            _transpose_block(rows_v.at[slot], tr_v.at[slot])
            start_stores(g, slot)

    wait_stores(0)
    wait_stores(1)


def kernel(token_ids, embedding_table):
    idx = jnp.transpose(token_ids).astype(jnp.int32)
    out_lin = _emb_lookup(idx, embedding_table)
    x = out_lin.reshape(SEQ, 4, GROUPS_PER_S, 8, G)
    x = jnp.transpose(x, (2, 4, 0, 1, 3))
    return x.reshape(BATCH, SEQ, DIM)


# speedup vs baseline: 1.5853x; 1.0909x over previous
import functools

import jax
import jax.numpy as jnp
from jax import lax
from jax.experimental import pallas as pl
from jax.experimental.pallas import tpu as pltpu
from jax.experimental.pallas import tpu_sc as plsc

NUM_EMB = 1000000
DIM = 32
BATCH = 16384
SEQ = 50
TOTAL = BATCH * SEQ

NC = 2
NS = 16
NW = NC * NS
PER_W = TOTAL // NW
G = 128
STEPS = PER_W // G
GROUPS_PER_S = BATCH // G
OUT_ROWS = TOTAL * DIM // G
TR_STRIDE = 131
NSLOT = 4

_mesh = plsc.VectorSubcoreMesh(core_axis_name="c", subcore_axis_name="s")


def _transpose_block(rows_v, tr_v):
    lanes = lax.iota(jnp.int32, 16)
    hi_lanes = lanes + 16

    @pl.loop(0, G, step=8)
    def _t(k0):
        for u in range(8):
            k = k0 + u
            kv = jnp.broadcast_to(k, (16,))
            v_lo = rows_v[k, pl.ds(0, 16)]
            v_hi = rows_v[k, pl.ds(16, 16)]
            plsc.store_scatter(tr_v, [lanes, kv], v_lo)
            plsc.store_scatter(tr_v, [hi_lanes, kv], v_hi)


@functools.partial(
    pl.kernel,
    out_type=jax.ShapeDtypeStruct((OUT_ROWS, G), jnp.float32),
    mesh=_mesh,
    scratch_types=[
        pltpu.VMEM((SEQ, BATCH // NW), jnp.int32),
        pltpu.VMEM((NSLOT, G, DIM), jnp.float32),
        pltpu.VMEM((NSLOT, DIM, TR_STRIDE), jnp.float32),
        pltpu.SemaphoreType.DMA,
        pltpu.SemaphoreType.DMA,
    ],
    compiler_params=pltpu.CompilerParams(
        use_tc_tiling_on_sc=False, needs_layout_passes=False
    ),
)
def _emb_lookup(idx_hbm, table_hbm, out_hbm, idx_v, rows_v, tr_v, gsem, ssem):
    wid = lax.axis_index("s") * NC + lax.axis_index("c")
    bw = BATCH // NW
    jpw = bw // G
    pltpu.sync_copy(idx_hbm.at[:, pl.ds(wid * bw, bw)], idx_v)

    def start_gather(i, slot):
        pltpu.async_copy(
            table_hbm.at[idx_v.at[i // jpw, pl.ds((i % jpw) * G, G)]],
            rows_v.at[slot],
            gsem,
        )

    def wait_gather(slot):
        pltpu.make_async_copy(
            table_hbm.at[idx_v.at[0, pl.ds(0, G)]], rows_v.at[slot], gsem
        ).wait()

    def start_stores(i, slot):
        s = i // jpw
        tb = wid * jpw + i % jpw
        base = s * 4096 + tb * 8
        for tf in range(4):
            pltpu.async_copy(
                tr_v.at[slot].at[pl.ds(8 * tf, 8), pl.ds(0, G)],
                out_hbm.at[pl.ds(base + tf * 1024, 8)],
                ssem,
            )

    def wait_stores(slot):
        for tf in range(4):
            pltpu.make_async_copy(
                tr_v.at[slot].at[pl.ds(8 * tf, 8), pl.ds(0, G)],
                out_hbm.at[pl.ds(0, 8)],
                ssem,
            ).wait()

    for slot in range(NSLOT - 1):
        start_gather(slot, slot)

    @pl.loop(0, STEPS, step=NSLOT)
    def _main(i):
        for slot in range(NSLOT):
            g = i + slot

            @pl.when(g + NSLOT - 1 < STEPS)
            def _():
                start_gather(g + NSLOT - 1, (slot + NSLOT - 1) % NSLOT)

            wait_gather(slot)

            @pl.when(g >= NSLOT)
            def _():
                wait_stores(slot)

            _transpose_block(rows_v.at[slot], tr_v.at[slot])
            start_stores(g, slot)

    for slot in range(NSLOT):
        wait_stores(slot)


def kernel(token_ids, embedding_table):
    idx = jnp.transpose(token_ids).astype(jnp.int32)
    out_lin = _emb_lookup(idx, embedding_table)
    x = out_lin.reshape(SEQ, 4, GROUPS_PER_S, 8, G)
    x = jnp.transpose(x, (2, 4, 0, 1, 3))
    return x.reshape(BATCH, SEQ, DIM)
